# baseline (speedup 1.0000x reference)
import functools

import jax
import jax.numpy as jnp
from jax import lax
from jax.experimental import pallas as pl
from jax.experimental.pallas import tpu as pltpu
from jax.experimental.pallas import tpu_sc as plsc

T = 2048
M = 256
N = 256
C = 64
TILE = 128
NT = T // TILE
NCAND = 128
NITEMS = 80

_F = jnp.float32


def _fiota(shape, dim):
    return lax.broadcasted_iota(jnp.int32, shape, dim).astype(_F)

def _route_body(inds_ref, pos_ref, tile_ref, cls_ref, lo_ref, hi_ref, first_ref):
    ids = inds_ref[:].astype(_F)
    O = jnp.where(ids == _fiota((T, C), 1), 1.0, 0.0)

    counts = jnp.sum(O, axis=0, keepdims=True)
    countsb = jnp.broadcast_to(counts, (C, C))
    LE = jnp.where(_fiota((C, C), 1) <= _fiota((C, C), 0), 1.0, 0.0)
    offs_incl_col = jnp.sum(LE * countsb, axis=1, keepdims=True)
    E64 = jnp.where(_fiota((C, C), 0) == _fiota((C, C), 1), 1.0, 0.0)
    counts_col = jnp.sum(E64 * countsb, axis=1, keepdims=True)
    offs_excl_col = offs_incl_col - counts_col

    def row64(xcol):
        return jnp.sum(E64 * jnp.broadcast_to(xcol, (C, C)), axis=0, keepdims=True)

    offs_excl = row64(offs_excl_col)
    offs_incl = row64(offs_incl_col)

    tril = jnp.where(_fiota((T, T), 1) <= _fiota((T, T), 0), 1.0, 0.0)
    Cincl = jnp.dot(tril, O, preferred_element_type=_F)
    pos = jnp.sum(O * (Cincl - 1.0 + offs_excl), axis=1, keepdims=True)
    pos_ref[:] = pos.astype(jnp.int32)

    r = _fiota((NCAND, 1), 0)
    P = jnp.where(r - 16.0 == _fiota((NCAND, C), 1), 1.0, 0.0)
    offs_pad = jnp.sum(P * jnp.broadcast_to(offs_excl, (NCAND, C)),
                       axis=1, keepdims=True)
    counts_pad = jnp.sum(P * jnp.broadcast_to(counts, (NCAND, C)),
                         axis=1, keepdims=True)
    offs_mod = offs_pad - jnp.floor(offs_pad / TILE) * TILE
    validc = (counts_pad > 0.0) & (offs_mod != 0.0)
    scand = jnp.where(r < float(NT), r * TILE,
                      jnp.where(validc, offs_pad, float(T) + r))

    E128 = jnp.where(_fiota((NCAND, NCAND), 0) == _fiota((NCAND, NCAND), 1),
                     1.0, 0.0)

    def row128(xcol):
        return jnp.sum(E128 * jnp.broadcast_to(xcol, (NCAND, NCAND)),
                       axis=0, keepdims=True)

    def bcast128(xrow):
        return jnp.broadcast_to(xrow, (NCAND, NCAND))

    scand_row = row128(scand)
    rank = jnp.sum(jnp.where(scand_row < scand, 1.0, 0.0), axis=1, keepdims=True)
    QT = jnp.where(row128(rank) == _fiota((NCAND, NCAND), 0), 1.0, 0.0)
    s = jnp.sum(QT * bcast128(scand_row), axis=1, keepdims=True)

    valid = s < float(T)
    tile = jnp.where(valid, jnp.floor(s / TILE), float(NT - 1))
    lo = jnp.where(valid, s - jnp.floor(s / TILE) * TILE, float(TILE))
    SH = jnp.where(_fiota((NCAND, NCAND), 1) == _fiota((NCAND, NCAND), 0) + 1.0,
                   1.0, 0.0)
    next_s = jnp.sum(SH * bcast128(row128(s)), axis=1, keepdims=True)
    next_tile = jnp.floor(next_s / TILE)
    hi = jnp.where((next_s < float(T)) & (next_tile == tile),
                   next_s - next_tile * TILE, float(TILE))
    sclamp = jnp.minimum(s, float(T - 1))
    cls = jnp.sum(jnp.where(jnp.broadcast_to(offs_incl, (NCAND, C)) <= sclamp,
                            1.0, 0.0), axis=1, keepdims=True)
    SHp = jnp.where(_fiota((NCAND, NCAND), 1) == _fiota((NCAND, NCAND), 0) - 1.0,
                    1.0, 0.0)
    prev_tile = jnp.sum(SHp * bcast128(row128(tile)), axis=1, keepdims=True)
    first = jnp.where((r == 0.0) | (tile != prev_tile), 1.0, 0.0)

    tile_ref[:] = tile.astype(jnp.int32)
    cls_ref[:] = cls.astype(jnp.int32)
    lo_ref[:] = lo.astype(jnp.int32)
    hi_ref[:] = hi.astype(jnp.int32)
    first_ref[:] = first.astype(jnp.int32)


def _route(inds2):
    shapes = ([jax.ShapeDtypeStruct((T, 1), jnp.int32)]
              + [jax.ShapeDtypeStruct((NCAND, 1), jnp.int32)] * 5)
    return pl.pallas_call(_route_body, out_shape=shapes)(inds2)


def _mega_body(tile_ref, cls_ref, lo_ref, hi_ref,
               pos_ref, x_ref, w_ref, b_ref, out_ref,
               xs_ref, outs_ref, wbuf_ref, sem_ref):
    GT = jnp.where(pos_ref[:] == lax.broadcasted_iota(jnp.int32, (T, T), 1),
                   1.0, 0.0).astype(jnp.bfloat16)

    x_bf = x_ref[:].astype(jnp.bfloat16)
    xs = lax.dot_general(GT, x_bf, (((0,), (0,)), ((), ())),
                         preferred_element_type=_F)
    xs_ref[:] = xs.astype(jnp.bfloat16)

    outs_ref[:] = jnp.zeros((T, N), _F)

    def start_dma(j, slot):
        return pltpu.make_async_copy(w_ref.at[cls_ref[j]], wbuf_ref.at[slot],
                                     sem_ref.at[slot])

    start_dma(0, 0).start()

    def item(j, carry):
        slot = lax.rem(j, 2)
        nslot = lax.rem(j + 1, 2)
        pltpu.make_async_copy(w_ref.at[cls_ref[j]], wbuf_ref.at[slot],
                              sem_ref.at[slot]).wait()

        @pl.when(j + 1 < NITEMS)
        def _prefetch():
            jn = jnp.minimum(j + 1, NITEMS - 1)
            start_dma(jn, nslot).start()

        tile = tile_ref[j]
        lo = lo_ref[j]
        hi = hi_ref[j]
        riota = lax.broadcasted_iota(jnp.int32, (TILE, 1), 0)
        mask = (riota >= lo) & (riota < hi)
        rows = xs_ref[pl.ds(tile * TILE, TILE), :]
        xm = jnp.where(mask, rows, jnp.bfloat16(0))
        wj = wbuf_ref[slot].astype(jnp.bfloat16)
        contrib = (jnp.dot(xm, wj, preferred_element_type=_F)
                   + jnp.where(mask, b_ref[cls_ref[j]], 0.0))
        outs_ref[pl.ds(tile * TILE, TILE), :] += contrib
        return carry

    lax.fori_loop(0, NITEMS, item, 0)

    out_ref[:] = jnp.dot(GT, outs_ref[:].astype(jnp.bfloat16),
                         preferred_element_type=_F)


def _mega(tile_t, cls_t, lo_t, hi_t, pos2, x, w, b):
    return pl.pallas_call(
        _mega_body,
        in_specs=[
            pl.BlockSpec(memory_space=pltpu.SMEM),
            pl.BlockSpec(memory_space=pltpu.SMEM),
            pl.BlockSpec(memory_space=pltpu.SMEM),
            pl.BlockSpec(memory_space=pltpu.SMEM),
            pl.BlockSpec(memory_space=pltpu.VMEM),
            pl.BlockSpec(memory_space=pltpu.VMEM),
            pl.BlockSpec(memory_space=pl.ANY),
            pl.BlockSpec(memory_space=pltpu.VMEM),
        ],
        out_specs=pl.BlockSpec(memory_space=pltpu.VMEM),
        out_shape=jax.ShapeDtypeStruct((T, N), jnp.float32),
        scratch_shapes=[
            pltpu.VMEM((T, M), jnp.bfloat16),
            pltpu.VMEM((T, N), jnp.float32),
            pltpu.VMEM((2, M, N), jnp.float32),
            pltpu.SemaphoreType.DMA((2,)),
        ],
    )(tile_t, cls_t, lo_t, hi_t, pos2, x, w, b)


def kernel(x, inds, w, b):
    inds2 = inds.astype(jnp.int32).reshape(T, 1)
    pos, tile_t, cls_t, lo_t, hi_t, first_t = _route(inds2)
    del first_t
    tables = [a.reshape(NCAND)[:NITEMS] for a in (tile_t, cls_t, lo_t, hi_t)]
    b2 = b.reshape(C, 1, N)
    return _mega(*tables, pos, x, w, b2)


# speedup vs baseline: 2.9837x; 2.9837x over previous
import functools

import jax
import jax.numpy as jnp
from jax import lax
from jax.experimental import pallas as pl
from jax.experimental.pallas import tpu as pltpu
from jax.experimental.pallas import tpu_sc as plsc

T = 2048
M = 256
N = 256
C = 64
TILE = 128
NT = T // TILE
NCAND = 128
NITEMS = 80
NBUF = 8

_F = jnp.float32


def _fiota(shape, dim):
    return lax.broadcasted_iota(jnp.int32, shape, dim).astype(_F)

def _route_body(inds_ref, pos_ref, tile_ref, cls_ref, lo_ref, hi_ref, first_ref):
    ids = inds_ref[:].astype(_F)
    O = jnp.where(ids == _fiota((T, C), 1), 1.0, 0.0)

    counts = jnp.sum(O, axis=0, keepdims=True)
    countsb = jnp.broadcast_to(counts, (C, C))
    LE = jnp.where(_fiota((C, C), 1) <= _fiota((C, C), 0), 1.0, 0.0)
    offs_incl_col = jnp.sum(LE * countsb, axis=1, keepdims=True)
    E64 = jnp.where(_fiota((C, C), 0) == _fiota((C, C), 1), 1.0, 0.0)
    counts_col = jnp.sum(E64 * countsb, axis=1, keepdims=True)
    offs_excl_col = offs_incl_col - counts_col

    def row64(xcol):
        return jnp.sum(E64 * jnp.broadcast_to(xcol, (C, C)), axis=0, keepdims=True)

    offs_excl = row64(offs_excl_col)
    offs_incl = row64(offs_incl_col)

    tril = jnp.where(_fiota((T, T), 1) <= _fiota((T, T), 0), 1.0, 0.0)
    Cincl = jnp.dot(tril, O, preferred_element_type=_F)
    pos = jnp.sum(O * (Cincl - 1.0 + offs_excl), axis=1, keepdims=True)
    pos_ref[:] = pos.astype(jnp.int32)

    r = _fiota((NCAND, 1), 0)
    P = jnp.where(r - 16.0 == _fiota((NCAND, C), 1), 1.0, 0.0)
    offs_pad = jnp.sum(P * jnp.broadcast_to(offs_excl, (NCAND, C)),
                       axis=1, keepdims=True)
    counts_pad = jnp.sum(P * jnp.broadcast_to(counts, (NCAND, C)),
                         axis=1, keepdims=True)
    offs_mod = offs_pad - jnp.floor(offs_pad / TILE) * TILE
    validc = (counts_pad > 0.0) & (offs_mod != 0.0)
    scand = jnp.where(r < float(NT), r * TILE,
                      jnp.where(validc, offs_pad, float(T) + r))

    E128 = jnp.where(_fiota((NCAND, NCAND), 0) == _fiota((NCAND, NCAND), 1),
                     1.0, 0.0)

    def row128(xcol):
        return jnp.sum(E128 * jnp.broadcast_to(xcol, (NCAND, NCAND)),
                       axis=0, keepdims=True)

    def bcast128(xrow):
        return jnp.broadcast_to(xrow, (NCAND, NCAND))

    scand_row = row128(scand)
    rank = jnp.sum(jnp.where(scand_row < scand, 1.0, 0.0), axis=1, keepdims=True)
    QT = jnp.where(row128(rank) == _fiota((NCAND, NCAND), 0), 1.0, 0.0)
    s = jnp.sum(QT * bcast128(scand_row), axis=1, keepdims=True)

    valid = s < float(T)
    tile = jnp.where(valid, jnp.floor(s / TILE), float(NT - 1))
    lo = jnp.where(valid, s - jnp.floor(s / TILE) * TILE, float(TILE))
    SH = jnp.where(_fiota((NCAND, NCAND), 1) == _fiota((NCAND, NCAND), 0) + 1.0,
                   1.0, 0.0)
    next_s = jnp.sum(SH * bcast128(row128(s)), axis=1, keepdims=True)
    next_tile = jnp.floor(next_s / TILE)
    hi = jnp.where((next_s < float(T)) & (next_tile == tile),
                   next_s - next_tile * TILE, float(TILE))
    sclamp = jnp.minimum(s, float(T - 1))
    cls = jnp.sum(jnp.where(jnp.broadcast_to(offs_incl, (NCAND, C)) <= sclamp,
                            1.0, 0.0), axis=1, keepdims=True)
    SHp = jnp.where(_fiota((NCAND, NCAND), 1) == _fiota((NCAND, NCAND), 0) - 1.0,
                    1.0, 0.0)
    prev_tile = jnp.sum(SHp * bcast128(row128(tile)), axis=1, keepdims=True)
    first = jnp.where((r == 0.0) | (tile != prev_tile), 1.0, 0.0)

    tile_ref[:] = tile.astype(jnp.int32)
    cls_ref[:] = cls.astype(jnp.int32)
    lo_ref[:] = lo.astype(jnp.int32)
    hi_ref[:] = hi.astype(jnp.int32)
    first_ref[:] = first.astype(jnp.int32)


def _route(inds2):
    shapes = ([jax.ShapeDtypeStruct((T, 1), jnp.int32)]
              + [jax.ShapeDtypeStruct((NCAND, 1), jnp.int32)] * 5)
    return pl.pallas_call(_route_body, out_shape=shapes)(inds2)


def _mega_body(tile_ref, cls_ref, lo_ref, hi_ref,
               pos_ref, x_ref, w_ref, b_ref, out_ref,
               xs_ref, outs_ref, wbuf_ref, sem_ref):
    GT = jnp.where(pos_ref[:] == lax.broadcasted_iota(jnp.int32, (T, T), 1),
                   1.0, 0.0).astype(jnp.bfloat16)

    x_bf = x_ref[:].astype(jnp.bfloat16)
    xs = lax.dot_general(GT, x_bf, (((0,), (0,)), ((), ())),
                         preferred_element_type=_F)
    xs_ref[:] = xs.astype(jnp.bfloat16)

    outs_ref[:] = jnp.zeros((T, N), _F)

    def start_dma(j, slot):
        return pltpu.make_async_copy(w_ref.at[cls_ref[j]], wbuf_ref.at[slot],
                                     sem_ref.at[slot])

    for k in range(NBUF):
        start_dma(k, k).start()

    def item(j, carry):
        slot = lax.rem(j, NBUF)
        pltpu.make_async_copy(w_ref.at[cls_ref[j]], wbuf_ref.at[slot],
                              sem_ref.at[slot]).wait()

        tile = tile_ref[j]
        lo = lo_ref[j]
        hi = hi_ref[j]
        riota = lax.broadcasted_iota(jnp.int32, (TILE, 1), 0)
        mask = (riota >= lo) & (riota < hi)
        rows = xs_ref[pl.ds(tile * TILE, TILE), :]
        xm = jnp.where(mask, rows, jnp.bfloat16(0))
        wj = wbuf_ref[slot].astype(jnp.bfloat16)
        contrib = (jnp.dot(xm, wj, preferred_element_type=_F)
                   + jnp.where(mask, b_ref[cls_ref[j]], 0.0))
        outs_ref[pl.ds(tile * TILE, TILE), :] += contrib

        @pl.when(j + NBUF < NITEMS)
        def _prefetch():
            jn = jnp.minimum(j + NBUF, NITEMS - 1)
            start_dma(jn, slot).start()

        return carry

    lax.fori_loop(0, NITEMS, item, 0)

    out_ref[:] = jnp.dot(GT, outs_ref[:].astype(jnp.bfloat16),
                         preferred_element_type=_F)


def _mega(tile_t, cls_t, lo_t, hi_t, pos2, x, w, b):
    return pl.pallas_call(
        _mega_body,
        in_specs=[
            pl.BlockSpec(memory_space=pltpu.SMEM),
            pl.BlockSpec(memory_space=pltpu.SMEM),
            pl.BlockSpec(memory_space=pltpu.SMEM),
            pl.BlockSpec(memory_space=pltpu.SMEM),
            pl.BlockSpec(memory_space=pltpu.VMEM),
            pl.BlockSpec(memory_space=pltpu.VMEM),
            pl.BlockSpec(memory_space=pl.ANY),
            pl.BlockSpec(memory_space=pltpu.VMEM),
        ],
        out_specs=pl.BlockSpec(memory_space=pltpu.VMEM),
        out_shape=jax.ShapeDtypeStruct((T, N), jnp.float32),
        scratch_shapes=[
            pltpu.VMEM((T, M), jnp.bfloat16),
            pltpu.VMEM((T, N), jnp.float32),
            pltpu.VMEM((NBUF, M, N), jnp.float32),
            pltpu.SemaphoreType.DMA((NBUF,)),
        ],
    )(tile_t, cls_t, lo_t, hi_t, pos2, x, w, b)


def kernel(x, inds, w, b):
    inds2 = inds.astype(jnp.int32).reshape(T, 1)
    pos, tile_t, cls_t, lo_t, hi_t, first_t = _route(inds2)
    del first_t
    tables = [a.reshape(NCAND)[:NITEMS] for a in (tile_t, cls_t, lo_t, hi_t)]
    b2 = b.reshape(C, 1, N)
    return _mega(*tables, pos, x, w, b2)


# speedup vs baseline: 3.1061x; 1.0410x over previous
import functools

import jax
import jax.numpy as jnp
from jax import lax
from jax.experimental import pallas as pl
from jax.experimental.pallas import tpu as pltpu
from jax.experimental.pallas import tpu_sc as plsc

T = 2048
M = 256
N = 256
C = 64
TILE = 128
NT = T // TILE
NCAND = 128
NITEMS = 80
NBUF = 8

_F = jnp.float32


def _fiota(shape, dim):
    return lax.broadcasted_iota(jnp.int32, shape, dim).astype(_F)

def _route_body(inds_ref, pos_ref, tile_ref, cls_ref, lo_ref, hi_ref):
    ids = inds_ref[:].astype(_F)
    O = jnp.where(ids == _fiota((T, C), 1), 1.0, 0.0)

    counts = jnp.sum(O, axis=0, keepdims=True)
    countsb = jnp.broadcast_to(counts, (C, C))
    LE = jnp.where(_fiota((C, C), 1) <= _fiota((C, C), 0), 1.0, 0.0)
    offs_incl_col = jnp.sum(LE * countsb, axis=1, keepdims=True)
    E64 = jnp.where(_fiota((C, C), 0) == _fiota((C, C), 1), 1.0, 0.0)
    counts_col = jnp.sum(E64 * countsb, axis=1, keepdims=True)
    offs_excl_col = offs_incl_col - counts_col

    def row64(xcol):
        return jnp.sum(E64 * jnp.broadcast_to(xcol, (C, C)), axis=0, keepdims=True)

    offs_excl = row64(offs_excl_col)
    offs_incl = row64(offs_incl_col)

    tril = jnp.where(_fiota((T, T), 1) <= _fiota((T, T), 0), 1.0, 0.0)
    Cincl = jnp.dot(tril, O, preferred_element_type=_F)
    pos = jnp.sum(O * (Cincl - 1.0 + offs_excl), axis=1, keepdims=True)
    pos_ref[:] = pos.astype(jnp.int32)

    r = _fiota((NCAND, 1), 0)
    P = jnp.where(r - 16.0 == _fiota((NCAND, C), 1), 1.0, 0.0)
    offs_pad = jnp.sum(P * jnp.broadcast_to(offs_excl, (NCAND, C)),
                       axis=1, keepdims=True)
    counts_pad = jnp.sum(P * jnp.broadcast_to(counts, (NCAND, C)),
                         axis=1, keepdims=True)
    offs_mod = offs_pad - jnp.floor(offs_pad / TILE) * TILE
    validc = (counts_pad > 0.0) & (offs_mod != 0.0)
    scand = jnp.where(r < float(NT), r * TILE,
                      jnp.where(validc, offs_pad, float(T) + r))

    E128 = jnp.where(_fiota((NCAND, NCAND), 0) == _fiota((NCAND, NCAND), 1),
                     1.0, 0.0)

    def row128(xcol):
        return jnp.sum(E128 * jnp.broadcast_to(xcol, (NCAND, NCAND)),
                       axis=0, keepdims=True)

    def bcast128(xrow):
        return jnp.broadcast_to(xrow, (NCAND, NCAND))

    scand_row = row128(scand)
    rank = jnp.sum(jnp.where(scand_row < scand, 1.0, 0.0), axis=1, keepdims=True)
    QT = jnp.where(row128(rank) == _fiota((NCAND, NCAND), 0), 1.0, 0.0)
    s = jnp.sum(QT * bcast128(scand_row), axis=1, keepdims=True)

    valid = s < float(T)
    tile = jnp.where(valid, jnp.floor(s / TILE), float(NT - 1))
    lo = jnp.where(valid, s - jnp.floor(s / TILE) * TILE, float(TILE))
    SH = jnp.where(_fiota((NCAND, NCAND), 1) == _fiota((NCAND, NCAND), 0) + 1.0,
                   1.0, 0.0)
    next_s = jnp.sum(SH * bcast128(row128(s)), axis=1, keepdims=True)
    next_tile = jnp.floor(next_s / TILE)
    hi = jnp.where((next_s < float(T)) & (next_tile == tile),
                   next_s - next_tile * TILE, float(TILE))
    sclamp = jnp.minimum(s, float(T - 1))
    cls = jnp.sum(jnp.where(jnp.broadcast_to(offs_incl, (NCAND, C)) <= sclamp,
                            1.0, 0.0), axis=1, keepdims=True)
    tile_ref[:] = tile[:NITEMS].astype(jnp.int32)
    cls_ref[:] = cls[:NITEMS].astype(jnp.int32)
    lo_ref[:] = lo[:NITEMS].astype(jnp.int32)
    hi_ref[:] = hi[:NITEMS].astype(jnp.int32)


def _route(inds2):
    shapes = ([jax.ShapeDtypeStruct((T, 1), jnp.int32)]
              + [jax.ShapeDtypeStruct((NITEMS, 1), jnp.int32)] * 4)
    return pl.pallas_call(_route_body, out_shape=shapes)(inds2)


def _mega_body(tile_ref, cls_ref, lo_ref, hi_ref,
               pos_ref, x_ref, w_ref, b_ref, out_ref,
               xs_ref, outs_ref, wbuf_ref, sem_ref):
    GT = jnp.where(pos_ref[:] == lax.broadcasted_iota(jnp.int32, (T, T), 1),
                   1.0, 0.0).astype(jnp.bfloat16)

    x_bf = x_ref[:].astype(jnp.bfloat16)
    xs = lax.dot_general(GT, x_bf, (((0,), (0,)), ((), ())),
                         preferred_element_type=_F)
    xs_ref[:] = xs.astype(jnp.bfloat16)

    outs_ref[:] = jnp.zeros((T, N), _F)

    def start_dma(j, slot):
        return pltpu.make_async_copy(w_ref.at[cls_ref[j, 0]], wbuf_ref.at[slot],
                                     sem_ref.at[slot])

    for k in range(NBUF):
        start_dma(k, k).start()

    def item(j, carry):
        slot = lax.rem(j, NBUF)
        pltpu.make_async_copy(w_ref.at[cls_ref[j, 0]], wbuf_ref.at[slot],
                              sem_ref.at[slot]).wait()

        tile = tile_ref[j, 0]
        lo = lo_ref[j, 0]
        hi = hi_ref[j, 0]
        riota = lax.broadcasted_iota(jnp.int32, (TILE, 1), 0)
        mask = (riota >= lo) & (riota < hi)
        rows = xs_ref[pl.ds(tile * TILE, TILE), :]
        xm = jnp.where(mask, rows, jnp.bfloat16(0))
        wj = wbuf_ref[slot].astype(jnp.bfloat16)
        contrib = (jnp.dot(xm, wj, preferred_element_type=_F)
                   + jnp.where(mask, b_ref[cls_ref[j, 0]], 0.0))
        outs_ref[pl.ds(tile * TILE, TILE), :] += contrib

        @pl.when(j + NBUF < NITEMS)
        def _prefetch():
            jn = jnp.minimum(j + NBUF, NITEMS - 1)
            start_dma(jn, slot).start()

        return carry

    lax.fori_loop(0, NITEMS, item, 0)

    out_ref[:] = jnp.dot(GT, outs_ref[:].astype(jnp.bfloat16),
                         preferred_element_type=_F)


def _mega(tile_t, cls_t, lo_t, hi_t, pos2, x, w, b):
    return pl.pallas_call(
        _mega_body,
        in_specs=[
            pl.BlockSpec(memory_space=pltpu.SMEM),
            pl.BlockSpec(memory_space=pltpu.SMEM),
            pl.BlockSpec(memory_space=pltpu.SMEM),
            pl.BlockSpec(memory_space=pltpu.SMEM),
            pl.BlockSpec(memory_space=pltpu.VMEM),
            pl.BlockSpec(memory_space=pltpu.VMEM),
            pl.BlockSpec(memory_space=pl.ANY),
            pl.BlockSpec(memory_space=pltpu.VMEM),
        ],
        out_specs=pl.BlockSpec(memory_space=pltpu.VMEM),
        out_shape=jax.ShapeDtypeStruct((T, N), jnp.float32),
        scratch_shapes=[
            pltpu.VMEM((T, M), jnp.bfloat16),
            pltpu.VMEM((T, N), jnp.float32),
            pltpu.VMEM((NBUF, M, N), jnp.float32),
            pltpu.SemaphoreType.DMA((NBUF,)),
        ],
    )(tile_t, cls_t, lo_t, hi_t, pos2, x, w, b)


def kernel(x, inds, w, b):
    inds2 = inds.astype(jnp.int32).reshape(T, 1)
    pos, tile_t, cls_t, lo_t, hi_t = _route(inds2)
    return _mega(tile_t, cls_t, lo_t, hi_t, pos, x, w, b.reshape(C, 1, N))
